# baseline (speedup 1.0000x reference)
import functools

import jax
import jax.numpy as jnp
from jax import lax
from jax.experimental import pallas as pl
from jax.experimental.pallas import tpu as pltpu
from jax.experimental.pallas import tpu_sc as plsc

_B = 16384
_F = 26
_D = 32
_XPAD = 128

_INFO = plsc.get_sparse_core_info()
_NC = _INFO.num_cores
_NS = _INFO.num_subcores
_NW = _NC * _NS
_BPW = _B // _NW
_NB = 64
_NCHUNK = _BPW // _NB
_NIDX = _NB * _F + 6

_NTOK = 1000000
_W = 256
_NSLABF = _NTOK // _W
_TREM = _NTOK - _NSLABF * _W
_SPAD = 31 * (_W + 5) + _W


def _tp_compute(spad_v, slab, out_v, width):
    stride = width + 5
    for r in range(_D):
        for g in range(width // 16):
            spad_v[pl.ds(r * stride + g * 16, 16)] = slab[r, pl.ds(g * 16, 16)]
    iota_lo = lax.iota(jnp.int32, 16) * stride
    iota_hi = iota_lo + 16 * stride

    def t_body(tq, _):
        t8 = tq * 8
        for j in range(8):
            lo = plsc.load_gather(spad_v, [iota_lo + (t8 + j)])
            hi = plsc.load_gather(spad_v, [iota_hi + (t8 + j)])
            out_v[pl.ds((t8 + j) * _D, 16)] = lo
            out_v[pl.ds((t8 + j) * _D + 16, 16)] = hi
        return 0

    lax.fori_loop(0, width // 8, t_body, 0)


_DEPTH = 2
_ROUNDS = -(-(-(-_NSLABF // _NW) + 1) // _DEPTH) + 1


def _tp_body(tabt_hbm, out_hbm, slab_a, slab_b, slab_c, slab_d, slab64_v,
             spad_v, out_a, out_b, out_c, out_d, out64_v,
             sem_ia, sem_ib, sem_ic, sem_id,
             sem_oa, sem_ob, sem_oc, sem_od, sem64):
    wid = lax.axis_index("s") * _NC + lax.axis_index("c")
    n_c = (_NSLABF - 1 - wid) // _NW + 1

    slabs = (slab_a, slab_b, slab_c, slab_d)
    outs = (out_a, out_b, out_c, out_d)
    sems_i = (sem_ia, sem_ib, sem_ic, sem_id)
    sems_o = (sem_oa, sem_ob, sem_oc, sem_od)

    def start_in(c, slab, sem):
        @pl.when(c < n_c)
        def _():
            tc = c * _NW + wid
            pltpu.async_copy(tabt_hbm.at[:, pl.ds(tc * _W, _W)], slab, sem)

    def wait_in(slab, sem):
        pltpu.make_async_copy(tabt_hbm.at[:, pl.ds(0, _W)], slab, sem).wait()

    def wait_out(out_v, sem):
        pltpu.make_async_copy(
            out_v, out_hbm.at[pl.ds(0, _W * _D)], sem).wait()

    def lane(p, c, slab, sem_i, out_v, sem_o):
        @pl.when(c < n_c)
        def _():
            wait_in(slab, sem_i)

            @pl.when(p > 0)
            def _w():
                wait_out(out_v, sem_o)

            _tp_compute(spad_v, slab, out_v, _W)
            tc = c * _NW + wid
            pltpu.async_copy(
                out_v, out_hbm.at[pl.ds(tc * _W * _D, _W * _D)], sem_o)
            start_in(c + _DEPTH, slab, sem_i)

    for j in range(_DEPTH):
        start_in(j, slabs[j], sems_i[j])

    def round_body(p, _):
        for j in range(_DEPTH):
            lane(p, _DEPTH * p + j, slabs[j], sems_i[j], outs[j], sems_o[j])
        return 0

    lax.fori_loop(0, _ROUNDS, round_body, 0)

    for j in range(_DEPTH):
        @pl.when(n_c >= j + 1)
        def _d(j=j):
            wait_out(outs[j], sems_o[j])

    @pl.when(wid == _NSLABF % _NW)
    def _rem():
        pltpu.async_copy(
            tabt_hbm.at[:, pl.ds(_NSLABF * _W, _TREM)], slab64_v, sem64).wait()
        _tp_compute(spad_v, slab64_v, out64_v, _TREM)
        pltpu.async_copy(
            out64_v, out_hbm.at[pl.ds(_NSLABF * _W * _D, _TREM * _D)],
            sem64).wait()


def _linearize_table(emb_weight):
    tab_t = emb_weight.T
    mesh = plsc.VectorSubcoreMesh(core_axis_name="c", subcore_axis_name="s")
    k = functools.partial(
        pl.kernel,
        mesh=mesh,
        out_type=jax.ShapeDtypeStruct((_NTOK * _D,), jnp.float32),
        scratch_types=(
            [pltpu.VMEM((_D, _W), jnp.float32)] * 4
            + [pltpu.VMEM((_D, _TREM), jnp.float32)]
            + [pltpu.VMEM((_SPAD,), jnp.float32)]
            + [pltpu.VMEM((_W * _D,), jnp.float32)] * 4
            + [pltpu.VMEM((_TREM * _D,), jnp.float32)]
            + [pltpu.SemaphoreType.DMA] * 9
        ),
        compiler_params=pltpu.CompilerParams(
            use_tc_tiling_on_sc=True, needs_layout_passes=False),
    )(_tp_body)
    return k(tab_t)


def _sc_body(x_hbm, tab_hbm, bias_hbm, out_hbm, xrow_v, idx_a, idx_b,
             rows_a, rows_b, out_v, bias_v, sem_a, sem_b, sem_o):
    wid = lax.axis_index("s") * _NC + lax.axis_index("c")
    pltpu.sync_copy(bias_hbm, bias_v)

    def stage_and_fire(c, idx_v, rows_v, sem):
        base = wid * _BPW + c * _NB
        pltpu.sync_copy(x_hbm.at[pl.ds(base, _NB), :], xrow_v)

        def pack_body(b, _):
            v1 = xrow_v[b, pl.ds(16, 16)]
            idx_v[pl.ds(b * _F + 16, 16)] = v1
            v0 = xrow_v[b, pl.ds(0, 16)]
            idx_v[pl.ds(b * _F, 16)] = v0
            return 0

        lax.fori_loop(0, _NB, pack_body, 0)
        return pltpu.async_copy(tab_hbm.at[idx_v], rows_v, sem)

    bufs = ((idx_a, rows_a, sem_a), (idx_b, rows_b, sem_b))
    gather = [None, None]
    gather[0] = stage_and_fire(0, *bufs[0])
    out_dma = None
    for c in range(_NCHUNK):
        p = c % 2
        gather[p].wait()
        if c + 1 < _NCHUNK:
            gather[1 - p] = stage_and_fire(c + 1, *bufs[1 - p])
        if out_dma is not None:
            out_dma.wait()
        rows_v = bufs[p][1]

        def row_body(b, _, rows_v=rows_v):
            rb = b * _F
            a0 = bias_v[pl.ds(0, 16)]
            a1 = bias_v[pl.ds(16, 16)]
            for f in range(_F):
                a0 = a0 + rows_v[rb + f, pl.ds(0, 16)]
                a1 = a1 + rows_v[rb + f, pl.ds(16, 16)]
            out_v[b, pl.ds(0, 16)] = a0
            out_v[b, pl.ds(16, 16)] = a1
            return 0

        lax.fori_loop(0, _NB, row_body, 0)
        base = wid * _BPW + c * _NB
        out_dma = pltpu.async_copy(out_v, out_hbm.at[pl.ds(base, _NB), :], sem_o)
    out_dma.wait()


def kernel(x, emb_weight, emb_bias):
    x_pad = jnp.pad(x.astype(jnp.int32), ((0, 0), (0, _XPAD - _F)))
    tab_lin = _linearize_table(emb_weight).reshape(_NTOK, _D)
    mesh = plsc.VectorSubcoreMesh(core_axis_name="c", subcore_axis_name="s")
    k = functools.partial(
        pl.kernel,
        mesh=mesh,
        out_type=jax.ShapeDtypeStruct((_B, _XPAD), jnp.float32),
        scratch_types=[
            pltpu.VMEM((_NB, _XPAD), jnp.int32),
            pltpu.VMEM((_NIDX,), jnp.int32),
            pltpu.VMEM((_NIDX,), jnp.int32),
            pltpu.VMEM((_NIDX, _D), jnp.float32),
            pltpu.VMEM((_NIDX, _D), jnp.float32),
            pltpu.VMEM((_NB, _XPAD), jnp.float32),
            pltpu.VMEM((_D,), jnp.float32),
            pltpu.SemaphoreType.DMA,
            pltpu.SemaphoreType.DMA,
            pltpu.SemaphoreType.DMA,
        ],
        compiler_params=pltpu.CompilerParams(use_tc_tiling_on_sc=False),
    )(_sc_body)
    out_pad = k(x_pad, tab_lin, emb_bias)
    return out_pad[:, :_D]


# speedup vs baseline: 1.0796x; 1.0796x over previous
import functools

import jax
import jax.numpy as jnp
from jax import lax
from jax.experimental import pallas as pl
from jax.experimental.pallas import tpu as pltpu
from jax.experimental.pallas import tpu_sc as plsc

_B = 16384
_F = 26
_D = 32
_XPAD = 128

_INFO = plsc.get_sparse_core_info()
_NC = _INFO.num_cores
_NS = _INFO.num_subcores
_NW = _NC * _NS
_BPW = _B // _NW
_NB = 64
_NCHUNK = _BPW // _NB
_NIDX = _NB * _F + 6

_NTOK = 1000000
_W = 128
_NSLABF = _NTOK // _W
_TREM = _NTOK - _NSLABF * _W
_SPAD = 31 * (_W + 5) + _W


def _tp_compute(spad_v, slab, out_v, width):
    stride = width + 5
    for r in range(_D):
        for g in range(width // 16):
            spad_v[pl.ds(r * stride + g * 16, 16)] = slab[r, pl.ds(g * 16, 16)]
    iota_lo = lax.iota(jnp.int32, 16) * stride
    iota_hi = iota_lo + 16 * stride

    def t_body(tq, _):
        t8 = tq * 8
        for j in range(8):
            lo = plsc.load_gather(spad_v, [iota_lo + (t8 + j)])
            hi = plsc.load_gather(spad_v, [iota_hi + (t8 + j)])
            out_v[pl.ds((t8 + j) * _D, 16)] = lo
            out_v[pl.ds((t8 + j) * _D + 16, 16)] = hi
        return 0

    lax.fori_loop(0, width // 8, t_body, 0)


_DEPTH = 2
_ROUNDS = -(-(-(-_NSLABF // _NW) + 1) // _DEPTH) + 1


def _tp_body(tabt_hbm, out_hbm, slab_a, slab_b, slab_c, slab_d, slab64_v,
             spad_v, out_a, out_b, out_c, out_d, out64_v,
             sem_ia, sem_ib, sem_ic, sem_id,
             sem_oa, sem_ob, sem_oc, sem_od, sem64):
    wid = lax.axis_index("s") * _NC + lax.axis_index("c")
    n_c = (_NSLABF - 1 - wid) // _NW + 1

    slabs = (slab_a, slab_b, slab_c, slab_d)
    outs = (out_a, out_b, out_c, out_d)
    sems_i = (sem_ia, sem_ib, sem_ic, sem_id)
    sems_o = (sem_oa, sem_ob, sem_oc, sem_od)

    def start_in(c, slab, sem):
        @pl.when(c < n_c)
        def _():
            tc = c * _NW + wid
            pltpu.async_copy(tabt_hbm.at[:, pl.ds(tc * _W, _W)], slab, sem)

    def wait_in(slab, sem):
        pltpu.make_async_copy(tabt_hbm.at[:, pl.ds(0, _W)], slab, sem).wait()

    def wait_out(out_v, sem):
        pltpu.make_async_copy(
            out_v, out_hbm.at[pl.ds(0, _W * _D)], sem).wait()

    def lane(p, c, slab, sem_i, out_v, sem_o):
        @pl.when(c < n_c)
        def _():
            wait_in(slab, sem_i)

            @pl.when(p > 0)
            def _w():
                wait_out(out_v, sem_o)

            _tp_compute(spad_v, slab, out_v, _W)
            tc = c * _NW + wid
            pltpu.async_copy(
                out_v, out_hbm.at[pl.ds(tc * _W * _D, _W * _D)], sem_o)
            start_in(c + _DEPTH, slab, sem_i)

    for j in range(_DEPTH):
        start_in(j, slabs[j], sems_i[j])

    def round_body(p, _):
        for j in range(_DEPTH):
            lane(p, _DEPTH * p + j, slabs[j], sems_i[j], outs[j], sems_o[j])
        return 0

    lax.fori_loop(0, _ROUNDS, round_body, 0)

    for j in range(_DEPTH):
        @pl.when(n_c >= j + 1)
        def _d(j=j):
            wait_out(outs[j], sems_o[j])

    @pl.when(wid == _NSLABF % _NW)
    def _rem():
        pltpu.async_copy(
            tabt_hbm.at[:, pl.ds(_NSLABF * _W, _TREM)], slab64_v, sem64).wait()
        _tp_compute(spad_v, slab64_v, out64_v, _TREM)
        pltpu.async_copy(
            out64_v, out_hbm.at[pl.ds(_NSLABF * _W * _D, _TREM * _D)],
            sem64).wait()


def _linearize_table(emb_weight):
    tab_t = emb_weight.T
    mesh = plsc.VectorSubcoreMesh(core_axis_name="c", subcore_axis_name="s")
    k = functools.partial(
        pl.kernel,
        mesh=mesh,
        out_type=jax.ShapeDtypeStruct((_NTOK * _D,), jnp.float32),
        scratch_types=(
            [pltpu.VMEM((_D, _W), jnp.float32)] * 4
            + [pltpu.VMEM((_D, _TREM), jnp.float32)]
            + [pltpu.VMEM((_SPAD,), jnp.float32)]
            + [pltpu.VMEM((_W * _D,), jnp.float32)] * 4
            + [pltpu.VMEM((_TREM * _D,), jnp.float32)]
            + [pltpu.SemaphoreType.DMA] * 9
        ),
        compiler_params=pltpu.CompilerParams(
            use_tc_tiling_on_sc=True, needs_layout_passes=False),
    )(_tp_body)
    return k(tab_t)


def _sc_body(x_hbm, tab_hbm, bias_hbm, out_hbm, xrow_v, idx_a, idx_b,
             rows_a, rows_b, out_v, bias_v, sem_a, sem_b, sem_o):
    wid = lax.axis_index("s") * _NC + lax.axis_index("c")
    pltpu.sync_copy(bias_hbm, bias_v)

    def stage_and_fire(c, idx_v, rows_v, sem):
        base = wid * _BPW + c * _NB
        pltpu.sync_copy(x_hbm.at[pl.ds(base, _NB), :], xrow_v)

        def pack_body(b, _):
            v1 = xrow_v[b, pl.ds(16, 16)]
            idx_v[pl.ds(b * _F + 16, 16)] = v1
            v0 = xrow_v[b, pl.ds(0, 16)]
            idx_v[pl.ds(b * _F, 16)] = v0
            return 0

        lax.fori_loop(0, _NB, pack_body, 0)
        return pltpu.async_copy(tab_hbm.at[idx_v], rows_v, sem)

    bufs = ((idx_a, rows_a, sem_a), (idx_b, rows_b, sem_b))
    gather = [None, None]
    gather[0] = stage_and_fire(0, *bufs[0])
    out_dma = None
    for c in range(_NCHUNK):
        p = c % 2
        gather[p].wait()
        if c + 1 < _NCHUNK:
            gather[1 - p] = stage_and_fire(c + 1, *bufs[1 - p])
        if out_dma is not None:
            out_dma.wait()
        rows_v = bufs[p][1]

        def row_body(b, _, rows_v=rows_v):
            rb = b * _F
            a0 = bias_v[pl.ds(0, 16)]
            a1 = bias_v[pl.ds(16, 16)]
            for f in range(_F):
                a0 = a0 + rows_v[rb + f, pl.ds(0, 16)]
                a1 = a1 + rows_v[rb + f, pl.ds(16, 16)]
            out_v[b, pl.ds(0, 16)] = a0
            out_v[b, pl.ds(16, 16)] = a1
            return 0

        lax.fori_loop(0, _NB, row_body, 0)
        base = wid * _BPW + c * _NB
        out_dma = pltpu.async_copy(out_v, out_hbm.at[pl.ds(base, _NB), :], sem_o)
    out_dma.wait()


def kernel(x, emb_weight, emb_bias):
    x_pad = jnp.pad(x.astype(jnp.int32), ((0, 0), (0, _XPAD - _F)))
    tab_lin = _linearize_table(emb_weight).reshape(_NTOK, _D)
    mesh = plsc.VectorSubcoreMesh(core_axis_name="c", subcore_axis_name="s")
    k = functools.partial(
        pl.kernel,
        mesh=mesh,
        out_type=jax.ShapeDtypeStruct((_B, _XPAD), jnp.float32),
        scratch_types=[
            pltpu.VMEM((_NB, _XPAD), jnp.int32),
            pltpu.VMEM((_NIDX,), jnp.int32),
            pltpu.VMEM((_NIDX,), jnp.int32),
            pltpu.VMEM((_NIDX, _D), jnp.float32),
            pltpu.VMEM((_NIDX, _D), jnp.float32),
            pltpu.VMEM((_NB, _XPAD), jnp.float32),
            pltpu.VMEM((_D,), jnp.float32),
            pltpu.SemaphoreType.DMA,
            pltpu.SemaphoreType.DMA,
            pltpu.SemaphoreType.DMA,
        ],
        compiler_params=pltpu.CompilerParams(use_tc_tiling_on_sc=False),
    )(_sc_body)
    out_pad = k(x_pad, tab_lin, emb_bias)
    return out_pad[:, :_D]


# speedup vs baseline: 1.1889x; 1.1013x over previous
import functools

import jax
import jax.numpy as jnp
from jax import lax
from jax.experimental import pallas as pl
from jax.experimental.pallas import tpu as pltpu
from jax.experimental.pallas import tpu_sc as plsc

_B = 16384
_F = 26
_D = 32
_XPAD = 128

_INFO = plsc.get_sparse_core_info()
_NC = _INFO.num_cores
_NS = _INFO.num_subcores
_NW = _NC * _NS
_BPW = _B // _NW
_NB = 64
_NCHUNK = _BPW // _NB
_NIDX = _NB * _F + 6

_NTOK = 1000000
_W = 128
_NSLABF = _NTOK // _W
_TREM = _NTOK - _NSLABF * _W
_SPAD = 31 * (_W + 5) + _W


def _tp_compute(spad_v, slab, out_v, width):
    stride = width + 5
    for r in range(_D):
        for g in range(width // 16):
            spad_v[pl.ds(r * stride + g * 16, 16)] = slab[r, pl.ds(g * 16, 16)]
    iota_lo = lax.iota(jnp.int32, 16) * stride
    iota_hi = iota_lo + 16 * stride

    def t_body(tq, _):
        t8 = tq * 8
        for j in range(8):
            lo = plsc.load_gather(spad_v, [iota_lo + (t8 + j)])
            hi = plsc.load_gather(spad_v, [iota_hi + (t8 + j)])
            out_v[pl.ds((t8 + j) * _D, 16)] = lo
            out_v[pl.ds((t8 + j) * _D + 16, 16)] = hi
        return 0

    lax.fori_loop(0, width // 8, t_body, 0)


_DEPTH = 3
_ROUNDS = -(-(-(-_NSLABF // _NW) + 1) // _DEPTH) + 1


def _tp_body(tabt_hbm, out_hbm, slab_a, slab_b, slab_c, slab_d, slab64_v,
             spad_v, out_a, out_b, out_c, out_d, out64_v,
             sem_ia, sem_ib, sem_ic, sem_id,
             sem_oa, sem_ob, sem_oc, sem_od, sem64):
    wid = lax.axis_index("s") * _NC + lax.axis_index("c")
    n_c = (_NSLABF - 1 - wid) // _NW + 1

    slabs = (slab_a, slab_b, slab_c, slab_d)
    outs = (out_a, out_b, out_c, out_d)
    sems_i = (sem_ia, sem_ib, sem_ic, sem_id)
    sems_o = (sem_oa, sem_ob, sem_oc, sem_od)

    def start_in(c, slab, sem):
        @pl.when(c < n_c)
        def _():
            tc = c * _NW + wid
            pltpu.async_copy(tabt_hbm.at[:, pl.ds(tc * _W, _W)], slab, sem)

    def wait_in(slab, sem):
        pltpu.make_async_copy(tabt_hbm.at[:, pl.ds(0, _W)], slab, sem).wait()

    def wait_out(out_v, sem):
        pltpu.make_async_copy(
            out_v, out_hbm.at[pl.ds(0, _W * _D)], sem).wait()

    def lane(p, c, slab, sem_i, out_v, sem_o):
        @pl.when(c < n_c)
        def _():
            wait_in(slab, sem_i)

            @pl.when(p > 0)
            def _w():
                wait_out(out_v, sem_o)

            _tp_compute(spad_v, slab, out_v, _W)
            tc = c * _NW + wid
            pltpu.async_copy(
                out_v, out_hbm.at[pl.ds(tc * _W * _D, _W * _D)], sem_o)
            start_in(c + _DEPTH, slab, sem_i)

    for j in range(_DEPTH):
        start_in(j, slabs[j], sems_i[j])

    def round_body(p, _):
        for j in range(_DEPTH):
            lane(p, _DEPTH * p + j, slabs[j], sems_i[j], outs[j], sems_o[j])
        return 0

    lax.fori_loop(0, _ROUNDS, round_body, 0)

    for j in range(_DEPTH):
        @pl.when(n_c >= j + 1)
        def _d(j=j):
            wait_out(outs[j], sems_o[j])

    @pl.when(wid == _NSLABF % _NW)
    def _rem():
        pltpu.async_copy(
            tabt_hbm.at[:, pl.ds(_NSLABF * _W, _TREM)], slab64_v, sem64).wait()
        _tp_compute(spad_v, slab64_v, out64_v, _TREM)
        pltpu.async_copy(
            out64_v, out_hbm.at[pl.ds(_NSLABF * _W * _D, _TREM * _D)],
            sem64).wait()


def _linearize_table(emb_weight):
    tab_t = emb_weight.T
    mesh = plsc.VectorSubcoreMesh(core_axis_name="c", subcore_axis_name="s")
    k = functools.partial(
        pl.kernel,
        mesh=mesh,
        out_type=jax.ShapeDtypeStruct((_NTOK * _D,), jnp.float32),
        scratch_types=(
            [pltpu.VMEM((_D, _W), jnp.float32)] * 4
            + [pltpu.VMEM((_D, _TREM), jnp.float32)]
            + [pltpu.VMEM((_SPAD,), jnp.float32)]
            + [pltpu.VMEM((_W * _D,), jnp.float32)] * 4
            + [pltpu.VMEM((_TREM * _D,), jnp.float32)]
            + [pltpu.SemaphoreType.DMA] * 9
        ),
        compiler_params=pltpu.CompilerParams(
            use_tc_tiling_on_sc=True, needs_layout_passes=False),
    )(_tp_body)
    return k(tab_t)


def _sc_body(x_hbm, tab_hbm, bias_hbm, out_hbm, xrow_v, idx_a, idx_b,
             rows_a, rows_b, out_v, bias_v, sem_a, sem_b, sem_o):
    wid = lax.axis_index("s") * _NC + lax.axis_index("c")
    pltpu.sync_copy(bias_hbm, bias_v)

    def stage_and_fire(c, idx_v, rows_v, sem):
        base = wid * _BPW + c * _NB
        pltpu.sync_copy(x_hbm.at[pl.ds(base, _NB), :], xrow_v)

        def pack_body(b, _):
            v1 = xrow_v[b, pl.ds(16, 16)]
            idx_v[pl.ds(b * _F + 16, 16)] = v1
            v0 = xrow_v[b, pl.ds(0, 16)]
            idx_v[pl.ds(b * _F, 16)] = v0
            return 0

        lax.fori_loop(0, _NB, pack_body, 0)
        return pltpu.async_copy(tab_hbm.at[idx_v], rows_v, sem)

    bufs = ((idx_a, rows_a, sem_a), (idx_b, rows_b, sem_b))
    gather = [None, None]
    gather[0] = stage_and_fire(0, *bufs[0])
    out_dma = None
    for c in range(_NCHUNK):
        p = c % 2
        gather[p].wait()
        if c + 1 < _NCHUNK:
            gather[1 - p] = stage_and_fire(c + 1, *bufs[1 - p])
        if out_dma is not None:
            out_dma.wait()
        rows_v = bufs[p][1]

        def row_body(b, _, rows_v=rows_v):
            rb = b * _F
            a0 = bias_v[pl.ds(0, 16)]
            a1 = bias_v[pl.ds(16, 16)]
            for f in range(_F):
                a0 = a0 + rows_v[rb + f, pl.ds(0, 16)]
                a1 = a1 + rows_v[rb + f, pl.ds(16, 16)]
            out_v[b, pl.ds(0, 16)] = a0
            out_v[b, pl.ds(16, 16)] = a1
            return 0

        lax.fori_loop(0, _NB, row_body, 0)
        base = wid * _BPW + c * _NB
        out_dma = pltpu.async_copy(out_v, out_hbm.at[pl.ds(base, _NB), :], sem_o)
    out_dma.wait()


def kernel(x, emb_weight, emb_bias):
    x_pad = jnp.pad(x.astype(jnp.int32), ((0, 0), (0, _XPAD - _F)))
    tab_lin = _linearize_table(emb_weight).reshape(_NTOK, _D)
    mesh = plsc.VectorSubcoreMesh(core_axis_name="c", subcore_axis_name="s")
    k = functools.partial(
        pl.kernel,
        mesh=mesh,
        out_type=jax.ShapeDtypeStruct((_B, _XPAD), jnp.float32),
        scratch_types=[
            pltpu.VMEM((_NB, _XPAD), jnp.int32),
            pltpu.VMEM((_NIDX,), jnp.int32),
            pltpu.VMEM((_NIDX,), jnp.int32),
            pltpu.VMEM((_NIDX, _D), jnp.float32),
            pltpu.VMEM((_NIDX, _D), jnp.float32),
            pltpu.VMEM((_NB, _XPAD), jnp.float32),
            pltpu.VMEM((_D,), jnp.float32),
            pltpu.SemaphoreType.DMA,
            pltpu.SemaphoreType.DMA,
            pltpu.SemaphoreType.DMA,
        ],
        compiler_params=pltpu.CompilerParams(use_tc_tiling_on_sc=False),
    )(_sc_body)
    out_pad = k(x_pad, tab_lin, emb_bias)
    return out_pad[:, :_D]
